# baseline (speedup 1.0000x reference)
import functools

import jax
import jax.numpy as jnp
from jax import lax
from jax.experimental import pallas as pl
from jax.experimental.pallas import tpu as pltpu
from jax.experimental.pallas import tpu_sc as plsc

NC, NS = 2, 16
NW = NC * NS
D = 64
V = 231
VP = 232
NB = 16384
NJ = 200
SPAN = NB // NW
NV = SPAN // 16
TBL = D * VP


def _emb_body(xT_hbm, tbl_hbm, out_hbm,
              tbl_v, idx_v, out_buf, isem0, isem1, osem0, osem1):
    wid = lax.axis_index("s") * NC + lax.axis_index("c")
    base = wid * SPAN
    isems = (isem0, isem1)
    osems = (osem0, osem1)

    pltpu.sync_copy(tbl_hbm, tbl_v)

    def idx_copy(j, buf):
        return pltpu.make_async_copy(
            xT_hbm.at[j, pl.ds(base, SPAN)], idx_v.at[buf], isems[buf])

    def out_copy(j, buf):
        return pltpu.make_async_copy(
            out_buf.at[buf],
            out_hbm.at[j, :, pl.ds(wid * (SPAN // 128), SPAN // 128)],
            osems[buf])

    idx_copy(0, 0).start()
    idx_copy(1, 1).start()

    def col(i, buf):
        j = i * 2 + buf
        idx_copy(j, buf).wait()
        pl.when(i > 0)(lambda: out_copy(j - 2, buf).wait())

        @plsc.parallel_loop(0, NV, unroll=8)
        def gathers(v):
            xq = idx_v[buf, pl.ds(v * 16, 16)]
            i0 = v // 8
            b0 = (v % 8) * 16
            for k in range(D):
                out_buf[buf, k // 8, i0, k % 8, pl.ds(b0, 16)] = (
                    plsc.load_gather(tbl_v.at[k], [xq]))
        out_copy(j, buf).start()
        pl.when(j + 2 < NJ)(lambda: idx_copy(j + 2, buf).start())

    def body(i, carry):
        col(i, 0)
        col(i, 1)
        return carry

    lax.fori_loop(0, NJ // 2, body, 0)
    out_copy(NJ - 2, 0).wait()
    out_copy(NJ - 1, 1).wait()


def kernel(x, table):
    xT = x.T.astype(jnp.int32)
    tbl = jnp.pad(table.T, ((0, 0), (0, VP - V)))

    mesh = plsc.VectorSubcoreMesh(core_axis_name="c", subcore_axis_name="s")
    run = pl.kernel(
        _emb_body,
        out_type=jax.ShapeDtypeStruct((NJ, D // 8, NB // 128, 8, 128),
                                      jnp.float32),
        mesh=mesh,
        scratch_types=[
            pltpu.VMEM((D, VP), jnp.float32),
            pltpu.VMEM((2, SPAN), jnp.int32),
            pltpu.VMEM((2, D // 8, SPAN // 128, 8, 128), jnp.float32),
            pltpu.SemaphoreType.DMA,
            pltpu.SemaphoreType.DMA,
            pltpu.SemaphoreType.DMA,
            pltpu.SemaphoreType.DMA,
        ],
        compiler_params=pltpu.CompilerParams(use_tc_tiling_on_sc=False,
                                             needs_layout_passes=False),
    )
    out5 = run(xT, tbl)
    return jnp.transpose(out5, (2, 4, 0, 1, 3)).reshape(NB, NJ, D)


# speedup vs baseline: 1.0473x; 1.0473x over previous
import functools

import jax
import jax.numpy as jnp
from jax import lax
from jax.experimental import pallas as pl
from jax.experimental.pallas import tpu as pltpu
from jax.experimental.pallas import tpu_sc as plsc

NC, NS = 2, 16
NW = NC * NS
D = 64
V = 231
VP = 232
NB = 16384
NJ = 200
SPAN = NB // NW
NV = SPAN // 16
TBL = D * VP


def _emb_body(xT_hbm, tbl_hbm, out_hbm,
              tbl_v, idx_v, out_buf, isem0, isem1, osem0, osem1):
    wid = lax.axis_index("s") * NC + lax.axis_index("c")
    base = wid * SPAN
    isems = (isem0, isem1)
    osems = (osem0, osem1)

    pltpu.sync_copy(tbl_hbm, tbl_v)

    def idx_copy(j, buf):
        return pltpu.make_async_copy(
            xT_hbm.at[j, pl.ds(base, SPAN)], idx_v.at[buf], isems[buf])

    def out_copy(j, buf):
        return pltpu.make_async_copy(
            out_buf.at[buf],
            out_hbm.at[j, :, pl.ds(wid * (SPAN // 128), SPAN // 128)],
            osems[buf])

    idx_copy(0, 0).start()
    idx_copy(1, 1).start()

    def col(i, buf):
        j = i * 2 + buf
        idx_copy(j, buf).wait()
        pl.when(i > 0)(lambda: out_copy(j - 2, buf).wait())

        @plsc.parallel_loop(0, NV, unroll=2)
        def gathers(v):
            xq = idx_v[buf, pl.ds(v * 16, 16)]
            i0 = v // 8
            b0 = (v % 8) * 16
            for k in range(D):
                out_buf[buf, k // 8, i0, k % 8, pl.ds(b0, 16)] = (
                    plsc.load_gather(tbl_v.at[k], [xq]))
        out_copy(j, buf).start()
        pl.when(j + 2 < NJ)(lambda: idx_copy(j + 2, buf).start())

    def body(i, carry):
        col(i, 0)
        col(i, 1)
        return carry

    lax.fori_loop(0, NJ // 2, body, 0)
    out_copy(NJ - 2, 0).wait()
    out_copy(NJ - 1, 1).wait()


def kernel(x, table):
    xT = x.T.astype(jnp.int32)
    tbl = jnp.pad(table.T, ((0, 0), (0, VP - V)))

    mesh = plsc.VectorSubcoreMesh(core_axis_name="c", subcore_axis_name="s")
    run = pl.kernel(
        _emb_body,
        out_type=jax.ShapeDtypeStruct((NJ, D // 8, NB // 128, 8, 128),
                                      jnp.float32),
        mesh=mesh,
        scratch_types=[
            pltpu.VMEM((D, VP), jnp.float32),
            pltpu.VMEM((2, SPAN), jnp.int32),
            pltpu.VMEM((2, D // 8, SPAN // 128, 8, 128), jnp.float32),
            pltpu.SemaphoreType.DMA,
            pltpu.SemaphoreType.DMA,
            pltpu.SemaphoreType.DMA,
            pltpu.SemaphoreType.DMA,
        ],
        compiler_params=pltpu.CompilerParams(use_tc_tiling_on_sc=False,
                                             needs_layout_passes=False),
    )
    out5 = run(xT, tbl)
    return jnp.transpose(out5, (2, 4, 0, 1, 3)).reshape(NB, NJ, D)


# speedup vs baseline: 1.1148x; 1.0644x over previous
import functools

import jax
import jax.numpy as jnp
from jax import lax
from jax.experimental import pallas as pl
from jax.experimental.pallas import tpu as pltpu
from jax.experimental.pallas import tpu_sc as plsc

NC, NS = 2, 16
NW = NC * NS
D = 64
V = 231
VP = 232
NB = 16384
NJ = 200
SPAN = NB // NW
NV = SPAN // 16
TBL = D * VP


def _emb_body(xT_hbm, tbl_hbm, out_hbm,
              tbl_v, idx_v, out_buf, isem0, isem1, osem0, osem1):
    wid = lax.axis_index("s") * NC + lax.axis_index("c")
    base = wid * SPAN
    isems = (isem0, isem1)
    osems = (osem0, osem1)

    pltpu.sync_copy(tbl_hbm, tbl_v)

    def idx_copy(j, buf):
        return pltpu.make_async_copy(
            xT_hbm.at[j, pl.ds(base, SPAN)], idx_v.at[buf], isems[buf])

    def out_copy(j, buf):
        return pltpu.make_async_copy(
            out_buf.at[buf],
            out_hbm.at[j, :, pl.ds(wid * (SPAN // 128), SPAN // 128)],
            osems[buf])

    idx_copy(0, 0).start()
    idx_copy(1, 1).start()

    def col(i, buf):
        j = i * 2 + buf
        idx_copy(j, buf).wait()
        pl.when(i > 0)(lambda: out_copy(j - 2, buf).wait())

        @plsc.parallel_loop(0, NV, unroll=4)
        def gathers(v):
            xq = idx_v[buf, pl.ds(v * 16, 16)]
            i0 = v // 8
            b0 = (v % 8) * 16
            for k in range(D):
                out_buf[buf, k // 8, i0, k % 8, pl.ds(b0, 16)] = (
                    plsc.load_gather(tbl_v.at[k], [xq]))
        out_copy(j, buf).start()
        pl.when(j + 2 < NJ)(lambda: idx_copy(j + 2, buf).start())

    def body(i, carry):
        col(i, 0)
        col(i, 1)
        return carry

    lax.fori_loop(0, NJ // 2, body, 0)
    out_copy(NJ - 2, 0).wait()
    out_copy(NJ - 1, 1).wait()


def kernel(x, table):
    xT = x.T.astype(jnp.int32)
    tbl = jnp.pad(table.T, ((0, 0), (0, VP - V)))

    mesh = plsc.VectorSubcoreMesh(core_axis_name="c", subcore_axis_name="s")
    run = pl.kernel(
        _emb_body,
        out_type=jax.ShapeDtypeStruct((NJ, D // 8, NB // 128, 8, 128),
                                      jnp.float32),
        mesh=mesh,
        scratch_types=[
            pltpu.VMEM((D, VP), jnp.float32),
            pltpu.VMEM((2, SPAN), jnp.int32),
            pltpu.VMEM((2, D // 8, SPAN // 128, 8, 128), jnp.float32),
            pltpu.SemaphoreType.DMA,
            pltpu.SemaphoreType.DMA,
            pltpu.SemaphoreType.DMA,
            pltpu.SemaphoreType.DMA,
        ],
        compiler_params=pltpu.CompilerParams(use_tc_tiling_on_sc=False,
                                             needs_layout_passes=False),
    )
    out5 = run(xT, tbl)
    return jnp.transpose(out5, (2, 4, 0, 1, 3)).reshape(NB, NJ, D)
